# baseline (speedup 1.0000x reference)
import functools

import jax
import jax.numpy as jnp
from jax import lax
from jax.experimental import pallas as pl
from jax.experimental.pallas import tpu as pltpu
from jax.experimental.pallas import tpu_sc as plsc

_NUM_CLASSES = 100000
_BATCH = 1024
_TINY = 1e-7

_NC = 2
_NS = 16
_L = 16
_NW = _NC * _NS
_BPW = _BATCH // _NW


def _make_sc_gather():
    mesh = plsc.VectorSubcoreMesh(core_axis_name="c", subcore_axis_name="s")

    @functools.partial(
        pl.kernel,
        mesh=mesh,
        out_type=jax.ShapeDtypeStruct((_BATCH,), jnp.float32),
        scratch_types=[
            pltpu.VMEM((_BPW,), jnp.int32),
            pltpu.VMEM((_BPW, 8, 128), jnp.float32),
            pltpu.VMEM((_BPW,), jnp.float32),
            pltpu.SemaphoreType.DMA,
        ],
    )
    def k(y_hbm, logits_t_hbm, out_hbm, y_v, tiles_v, val_v, sem):
        wid = lax.axis_index("s") * _NC + lax.axis_index("c")
        base = pl.multiple_of(wid * _BPW, _BPW)
        colblk = (base // 128) * 128
        pltpu.sync_copy(y_hbm.at[pl.ds(base, _BPW)], y_v)
        lane = lax.iota(jnp.int32, _L)
        copies = []
        for c in range(_BPW // _L):
            yc = y_v[pl.ds(c * _L, _L)]
            for j in range(_L):
                r = c * _L + j
                rowblk = (yc[j] // 8) * 8
                copies.append(
                    pltpu.async_copy(
                        logits_t_hbm.at[pl.ds(rowblk, 8), pl.ds(colblk, 128)],
                        tiles_v.at[r], sem))
        for cp in copies:
            cp.wait()
        for c in range(_BPW // _L):
            yc = y_v[pl.ds(c * _L, _L)]
            res = jnp.zeros((_L,), jnp.float32)
            for j in range(_L):
                r = c * _L + j
                sub = yc[j] % 8
                col = base - colblk + r
                s16 = (col // _L) * _L
                v16 = tiles_v[r, sub, pl.ds(s16, _L)]
                res = jnp.where(lane == j, v16[r % _L], res)
            val_v[pl.ds(c * _L, _L)] = res
        pltpu.sync_copy(val_v, out_hbm.at[pl.ds(base, _BPW)])

    return k


_sc_gather = _make_sc_gather()


def _tc_logmean_body(x_ref, o_ref):
    o_ref[0, 0] = -jnp.mean(jnp.log(x_ref[...] + _TINY))


_tc_logmean = pl.pallas_call(
    _tc_logmean_body,
    out_shape=jax.ShapeDtypeStruct((1, 1), jnp.float32),
    out_specs=pl.BlockSpec(memory_space=pltpu.SMEM),
)


def kernel(y, logits):
    vals = _sc_gather(y, logits.T)
    return _tc_logmean(vals.reshape(8, 128))[0, 0]


# speedup vs baseline: 1.0387x; 1.0387x over previous
import functools

import jax
import jax.numpy as jnp
from jax import lax
from jax.experimental import pallas as pl
from jax.experimental.pallas import tpu as pltpu
from jax.experimental.pallas import tpu_sc as plsc

_NUM_CLASSES = 100000
_BATCH = 1024
_TINY = 1e-7

_NC = 2
_NS = 16
_L = 16
_NW = _NC * _NS
_BPW = _BATCH // _NW


def _make_sc_gather():
    mesh = plsc.VectorSubcoreMesh(core_axis_name="c", subcore_axis_name="s")

    @functools.partial(
        pl.kernel,
        mesh=mesh,
        out_type=jax.ShapeDtypeStruct((_BATCH,), jnp.float32),
        scratch_types=[
            pltpu.VMEM((_BPW,), jnp.int32),
            pltpu.VMEM((_BPW, _BATCH), jnp.float32),
            pltpu.VMEM((_BPW,), jnp.float32),
            pltpu.SemaphoreType.DMA,
        ],
    )
    def k(y_hbm, logits_t_hbm, out_hbm, y_v, rows_v, val_v, sem):
        wid = lax.axis_index("s") * _NC + lax.axis_index("c")
        base = pl.multiple_of(wid * _BPW, _BPW)
        pltpu.sync_copy(y_hbm.at[pl.ds(base, _BPW)], y_v)
        lane = lax.iota(jnp.int32, _L)
        pltpu.async_copy(logits_t_hbm.at[y_v], rows_v, sem).wait()
        for c in range(_BPW // _L):
            s16 = base + c * _L
            res = jnp.zeros((_L,), jnp.float32)
            for j in range(_L):
                r = c * _L + j
                v16 = rows_v[r, pl.ds(s16, _L)]
                res = jnp.where(lane == j, v16[j], res)
            val_v[pl.ds(c * _L, _L)] = res
        pltpu.sync_copy(val_v, out_hbm.at[pl.ds(base, _BPW)])

    return k


_sc_gather = _make_sc_gather()


def _tc_logmean_body(x_ref, o_ref):
    o_ref[0, 0] = -jnp.mean(jnp.log(x_ref[...] + _TINY))


_tc_logmean = pl.pallas_call(
    _tc_logmean_body,
    out_shape=jax.ShapeDtypeStruct((1, 1), jnp.float32),
    out_specs=pl.BlockSpec(memory_space=pltpu.SMEM),
)


def kernel(y, logits):
    vals = _sc_gather(y, logits.T)
    return _tc_logmean(vals.reshape(8, 128))[0, 0]
